# baseline (speedup 1.0000x reference)
import math

import jax
import jax.numpy as jnp
from jax import lax
from jax.experimental import pallas as pl
from jax.experimental.pallas import tpu as pltpu
from jax.experimental.pallas import tpu_sc as plsc

D_MODEL = 64
SCALE = math.sqrt(D_MODEL)

NC = 2
NS = 16
NW = NC * NS

L = 16
BLK = 128
T_DIM = 200
B_DIM = 4096
NBLK = T_DIM * B_DIM // BLK
BPW = NBLK // NW
BH = B_DIM // BLK


def _emb_body(idx_hbm, tab_hbm, out_hbm, idx_all, rows_v, trs_v, sem_g, sem_w):
    c = lax.axis_index("c")
    s = lax.axis_index("s")
    wid = s * NC + c
    g0 = wid * BPW

    pltpu.sync_copy(idx_hbm.at[pl.ds(g0, BPW)], idx_all)

    def gather(gl, slot):
        return pltpu.make_async_copy(
            tab_hbm.at[idx_all.at[gl]], rows_v.at[slot], sem_g)

    def out_writes(gl, slot):
        g = g0 + gl
        t = g // BH
        bb = g - t * BH
        return [
            pltpu.make_async_copy(
                trs_v.at[slot, dh], out_hbm.at[t, dh, bb], sem_w)
            for dh in range(8)
        ]

    gather(0, 0).start()

    def loop_body(gl, carry):
        cur = lax.rem(gl, 2)
        nxt = 1 - cur

        @pl.when(gl < BPW - 1)
        def _():
            gather(gl + 1, nxt).start()

        gather(gl, cur).wait()

        @pl.when(gl >= 2)
        def _():
            for cp in out_writes(gl - 2, cur):
                cp.wait()

        def dh_body(dh, cc):
            for dl in range(8):
                dvec = jnp.full((L,), dh * 8 + dl, jnp.int32)
                for k in range(8):
                    bidx = k * L + lax.iota(jnp.int32, L)
                    v = plsc.load_gather(rows_v.at[cur], [bidx, dvec])
                    trs_v[cur, dh, dl, pl.ds(k * L, L)] = v * SCALE
            return cc

        lax.fori_loop(0, 8, dh_body, 0)

        for cp in out_writes(gl, cur):
            cp.start()
        return carry

    lax.fori_loop(0, BPW, loop_body, 0)

    for gl in (BPW - 2, BPW - 1):
        for cp in out_writes(gl, gl % 2):
            cp.wait()


@jax.jit
def _emb(xf2, table):
    mesh = plsc.VectorSubcoreMesh(
        core_axis_name="c", subcore_axis_name="s",
        num_cores=NC, num_subcores=NS)
    f = pl.kernel(
        _emb_body,
        out_type=jax.ShapeDtypeStruct((T_DIM, 8, BH, 8, BLK), jnp.float32),
        mesh=mesh,
        scratch_types=[
            pltpu.VMEM((BPW, BLK), jnp.int32),
            pltpu.VMEM((2, BLK, D_MODEL), jnp.float32),
            pltpu.VMEM((2, 8, 8, BLK), jnp.float32),
            pltpu.SemaphoreType.DMA,
            pltpu.SemaphoreType.DMA,
        ],
        compiler_params=pltpu.CompilerParams(
            use_tc_tiling_on_sc=False, needs_layout_passes=False),
    )
    return f(xf2, table)


def kernel(x, table):
    b, t = x.shape
    assert (b, t) == (B_DIM, T_DIM)
    xf2 = x.T.reshape(NBLK, BLK).astype(jnp.int32)
    out5 = _emb(xf2, table)
    out = out5.transpose(2, 4, 0, 1, 3).reshape(B_DIM, T_DIM, D_MODEL)
    return out


# speedup vs baseline: 1.2771x; 1.2771x over previous
import math

import jax
import jax.numpy as jnp
from jax import lax
from jax.experimental import pallas as pl
from jax.experimental.pallas import tpu as pltpu
from jax.experimental.pallas import tpu_sc as plsc

D_MODEL = 64
SCALE = math.sqrt(D_MODEL)

NC = 2
NS = 16
NW = NC * NS

L = 16
BLK = 128
T_DIM = 200
B_DIM = 4096
NBLK = T_DIM * B_DIM // BLK
BPW = NBLK // NW
BH = B_DIM // BLK
KV = BLK // L
VOC_PAIR = 500000


def _emb_body(idx_hbm, tab_hbm, out_hbm,
              idx_all, rows_v, trs_v, sem_g, sem_w):
    c = lax.axis_index("c")
    s = lax.axis_index("s")
    wid = s * NC + c
    g0 = wid * BPW

    pltpu.sync_copy(idx_hbm.at[pl.ds(g0, BPW)], idx_all)

    def gather(gl, slot):
        return pltpu.make_async_copy(
            tab_hbm.at[idx_all.at[gl]], rows_v.at[slot], sem_g)

    def out_writes(gl, slot):
        g = g0 + gl
        t = g // BH
        bb = g - t * BH
        return [
            pltpu.make_async_copy(
                trs_v.at[slot, dh], out_hbm.at[t, dh, bb], sem_w)
            for dh in range(8)
        ]

    def transpose(slot):
        @plsc.parallel_loop(0, 8, unroll=2)
        def _dh(dh):
            for dl in range(8):
                dvec = jnp.full((L,), dh * 8 + dl, jnp.int32)
                for k in range(KV):
                    bidx = k * L + lax.iota(jnp.int32, L)
                    v = plsc.load_gather(rows_v.at[slot], [bidx, dvec])
                    trs_v[slot, dh, dl, pl.ds(k * L, L)] = v * SCALE

    def half(gl, slot, nslot):
        @pl.when(gl + 1 < BPW)
        def _():
            gather(gl + 1, nslot).start()

        gather(gl, slot).wait()

        @pl.when(gl >= 2)
        def _():
            for cp in out_writes(gl - 2, slot):
                cp.wait()

        transpose(slot)
        for cp in out_writes(gl, slot):
            cp.start()

    gather(0, 0).start()

    def loop_body(i, carry):
        half(2 * i, 0, 1)
        half(2 * i + 1, 1, 0)
        return carry

    lax.fori_loop(0, BPW // 2, loop_body, 0)

    for cp in out_writes(BPW - 2, 0):
        cp.wait()
    for cp in out_writes(BPW - 1, 1):
        cp.wait()


@jax.jit
def _emb(xf2, tab2):
    mesh = plsc.VectorSubcoreMesh(
        core_axis_name="c", subcore_axis_name="s",
        num_cores=NC, num_subcores=NS)
    f = pl.kernel(
        _emb_body,
        out_type=jax.ShapeDtypeStruct((T_DIM, 8, BH, 8, BLK), jnp.float32),
        mesh=mesh,
        scratch_types=[
            pltpu.VMEM((BPW, BLK), jnp.int32),
            pltpu.VMEM((2, BLK, D_MODEL), jnp.float32),
            pltpu.VMEM((2, 8, 8, BLK), jnp.float32),
            pltpu.SemaphoreType.DMA,
            pltpu.SemaphoreType.DMA,
        ],
        compiler_params=pltpu.CompilerParams(
            use_tc_tiling_on_sc=False, needs_layout_passes=False),
    )
    return f(xf2, tab2)


def kernel(x, table):
    b, t = x.shape
    assert (b, t) == (B_DIM, T_DIM)
    xf2 = x.T.reshape(NBLK, BLK).astype(jnp.int32)
    out5 = _emb(xf2, table)
    out = out5.transpose(2, 4, 0, 1, 3).reshape(B_DIM, T_DIM, D_MODEL)
    return out


# speedup vs baseline: 1.5422x; 1.2076x over previous
import math

import jax
import jax.numpy as jnp
from jax import lax
from jax.experimental import pallas as pl
from jax.experimental.pallas import tpu as pltpu
from jax.experimental.pallas import tpu_sc as plsc

D_MODEL = 64
SCALE = math.sqrt(D_MODEL)

NC = 2
NS = 16
NW = NC * NS

L = 16
BLK = 128
T_DIM = 200
B_DIM = 4096
NBLK = T_DIM * B_DIM // BLK
BPW = NBLK // NW
BH = B_DIM // BLK
KV = BLK // L
VOC_PAIR = 500000


def _emb_body(idx_hbm, tab_hbm, out_hbm,
              idx_all, rows_v, trs_v, sem_g, sem_w):
    c = lax.axis_index("c")
    s = lax.axis_index("s")
    wid = s * NC + c
    g0 = wid * BPW

    pltpu.sync_copy(idx_hbm.at[pl.ds(g0, BPW)], idx_all)

    def gather(gl, slot):
        return pltpu.make_async_copy(
            tab_hbm.at[idx_all.at[gl]], rows_v.at[slot], sem_g)

    def out_writes(gl, slot):
        g = g0 + gl
        t = g // BH
        bb = g - t * BH
        return [
            pltpu.make_async_copy(
                trs_v.at[slot, pl.ds(dh * 8, 8)], out_hbm.at[t, dh, bb],
                sem_w)
            for dh in range(8)
        ]

    def transpose(slot):
        rows = rows_v.at[slot]
        tflat = trs_v.at[slot]

        @plsc.parallel_loop(0, D_MODEL, unroll=2)
        def _d(d):
            dvec = jnp.full((L,), d, jnp.int32)
            for k in range(KV):
                bidx = k * L + lax.iota(jnp.int32, L)
                v = plsc.load_gather(rows, [bidx, dvec])
                tflat[d, pl.ds(k * L, L)] = v * SCALE

    def half(gl, slot, nslot):
        @pl.when(gl + 1 < BPW)
        def _():
            gather(gl + 1, nslot).start()

        gather(gl, slot).wait()

        @pl.when(gl >= 2)
        def _():
            for cp in out_writes(gl - 2, slot):
                cp.wait()

        transpose(slot)
        for cp in out_writes(gl, slot):
            cp.start()

    gather(0, 0).start()

    def loop_body(i, carry):
        half(2 * i, 0, 1)
        half(2 * i + 1, 1, 0)
        return carry

    lax.fori_loop(0, BPW // 2, loop_body, 0)

    for cp in out_writes(BPW - 2, 0):
        cp.wait()
    for cp in out_writes(BPW - 1, 1):
        cp.wait()


@jax.jit
def _emb(xf2, tab2):
    mesh = plsc.VectorSubcoreMesh(
        core_axis_name="c", subcore_axis_name="s",
        num_cores=NC, num_subcores=NS)
    f = pl.kernel(
        _emb_body,
        out_type=jax.ShapeDtypeStruct((T_DIM, 8, BH, 8, BLK), jnp.float32),
        mesh=mesh,
        scratch_types=[
            pltpu.VMEM((BPW, BLK), jnp.int32),
            pltpu.VMEM((2, BLK, D_MODEL), jnp.float32),
            pltpu.VMEM((2, D_MODEL, BLK), jnp.float32),
            pltpu.SemaphoreType.DMA,
            pltpu.SemaphoreType.DMA,
        ],
        compiler_params=pltpu.CompilerParams(
            use_tc_tiling_on_sc=False, needs_layout_passes=False),
    )
    return f(xf2, tab2)


def kernel(x, table):
    b, t = x.shape
    assert (b, t) == (B_DIM, T_DIM)
    xf2 = x.T.reshape(NBLK, BLK).astype(jnp.int32)
    out5 = _emb(xf2, table)
    out = out5.transpose(2, 4, 0, 1, 3).reshape(B_DIM, T_DIM, D_MODEL)
    return out


# speedup vs baseline: 2.4640x; 1.5978x over previous
import math

import jax
import jax.numpy as jnp
from jax import lax
from jax.experimental import pallas as pl
from jax.experimental.pallas import tpu as pltpu
from jax.experimental.pallas import tpu_sc as plsc

D_MODEL = 64
SCALE = math.sqrt(D_MODEL)

NC = 2
NS = 16
NW = NC * NS

L = 16
BLK = 128
T_DIM = 200
B_DIM = 4096
NBLK = T_DIM * B_DIM // BLK
BPW = NBLK // NW
BH = B_DIM // BLK
KV = BLK // L
VOC_PAIR = 500000


def _emb_body(idx_hbm, tab_hbm, out_hbm,
              idx_all, rows_v, trs_v, sem_g, sem_w):
    c = lax.axis_index("c")
    s = lax.axis_index("s")
    wid = s * NC + c
    g0 = wid * BPW

    pltpu.sync_copy(idx_hbm.at[pl.ds(g0, BPW)], idx_all)

    def gather(gl, slot):
        return pltpu.make_async_copy(
            tab_hbm.at[idx_all.at[gl]], rows_v.at[slot], sem_g)

    def out_writes(gl, slot):
        g = g0 + gl
        t = g // BH
        bb = g - t * BH
        return [
            pltpu.make_async_copy(
                trs_v.at[slot, pl.ds(dh * 8, 8), pl.ds(0, BLK)],
                out_hbm.at[t, dh, bb], sem_w)
            for dh in range(8)
        ]

    def transpose(slot):
        rows = rows_v.at[slot]
        tp = trs_v.at[slot]

        @plsc.parallel_loop(0, BLK, unroll=2)
        def _b(b):
            bvec = jnp.full((L,), b, jnp.int32)
            for dblk in range(D_MODEL // L):
                didx = dblk * L + lax.iota(jnp.int32, L)
                v = rows[b, pl.ds(dblk * L, L)]
                plsc.store_scatter(tp, [didx, bvec], v * SCALE)

    def half(gl, slot, nslot):
        @pl.when(gl + 1 < BPW)
        def _():
            gather(gl + 1, nslot).start()

        gather(gl, slot).wait()

        @pl.when(gl >= 2)
        def _():
            for cp in out_writes(gl - 2, slot):
                cp.wait()

        transpose(slot)
        for cp in out_writes(gl, slot):
            cp.start()

    gather(0, 0).start()

    def loop_body(i, carry):
        half(2 * i, 0, 1)
        half(2 * i + 1, 1, 0)
        return carry

    lax.fori_loop(0, BPW // 2, loop_body, 0)

    for cp in out_writes(BPW - 2, 0):
        cp.wait()
    for cp in out_writes(BPW - 1, 1):
        cp.wait()


@jax.jit
def _emb(xf2, tab2):
    mesh = plsc.VectorSubcoreMesh(
        core_axis_name="c", subcore_axis_name="s",
        num_cores=NC, num_subcores=NS)
    f = pl.kernel(
        _emb_body,
        out_type=jax.ShapeDtypeStruct((T_DIM, 8, BH, 8, BLK), jnp.float32),
        mesh=mesh,
        scratch_types=[
            pltpu.VMEM((BPW, BLK), jnp.int32),
            pltpu.VMEM((2, BLK, D_MODEL), jnp.float32),
            pltpu.VMEM((2, D_MODEL, BLK + 1), jnp.float32),
            pltpu.SemaphoreType.DMA,
            pltpu.SemaphoreType.DMA,
        ],
        compiler_params=pltpu.CompilerParams(
            use_tc_tiling_on_sc=False, needs_layout_passes=False),
    )
    return f(xf2, tab2)


def kernel(x, table):
    b, t = x.shape
    assert (b, t) == (B_DIM, T_DIM)
    xf2 = x.T.reshape(NBLK, BLK).astype(jnp.int32)
    out5 = _emb(xf2, table)
    out = out5.transpose(2, 4, 0, 1, 3).reshape(B_DIM, T_DIM, D_MODEL)
    return out
